# baseline (speedup 1.0000x reference)
import functools

import jax
import jax.numpy as jnp
from jax import lax
from jax.experimental import pallas as pl
from jax.experimental.pallas import tpu as pltpu
from jax.experimental.pallas import tpu_sc as plsc

VOCAB = 1000000
D = 64
B = 1024
S = 200

NC = 2
NS = 16
NW = NC * NS
ROWS_PER_W = (B * S) // NW
CHUNK = 128
NCHUNK = ROWS_PER_W // CHUNK


def _pos_encoding_2x() -> jax.Array:
    depth = D // 2
    positions = jnp.arange(S)[:, None].astype(jnp.float32)
    depths = jnp.arange(depth, dtype=jnp.float32)[None, :] / depth
    angle_rates = 1.0 / (10000.0 ** depths)
    angle_rads = positions * angle_rates
    pos = jnp.concatenate([jnp.sin(angle_rads), jnp.cos(angle_rads)], axis=-1)
    pos = pos.astype(jnp.float32)
    return jnp.concatenate([pos, pos], axis=0)


def _sc_body(table_hbm, idx_hbm, pos_hbm, out_hbm, idx_v, rows_v, pos_v, sem):
    w = lax.axis_index("s") * NC + lax.axis_index("c")
    pltpu.sync_copy(idx_hbm.at[w], idx_v)
    pltpu.sync_copy(pos_hbm, pos_v)

    def chunk_body(c, _):
        pltpu.async_copy(table_hbm.at[idx_v.at[c]], rows_v, sem).wait()
        off = lax.rem(c * CHUNK, S)

        def row_body(s, _):
            p = off + s
            for d in range(D // 16):
                sl = pl.ds(16 * d, 16)
                rows_v[s, sl] = rows_v[s, sl] * 8.0 + pos_v[p, sl]
            return 0

        lax.fori_loop(0, CHUNK, row_body, 0)
        pltpu.sync_copy(rows_v, out_hbm.at[pl.ds(w * ROWS_PER_W + c * CHUNK, CHUNK)])
        return 0

    lax.fori_loop(0, NCHUNK, chunk_body, 0)


@jax.jit
def _embed(table, idx, pos2):
    mesh = plsc.VectorSubcoreMesh(core_axis_name="c", subcore_axis_name="s")
    k = functools.partial(
        pl.kernel,
        out_type=jax.ShapeDtypeStruct((B * S, D), jnp.float32),
        mesh=mesh,
        scratch_types=[
            pltpu.VMEM((NCHUNK, CHUNK), jnp.int32),
            pltpu.VMEM((CHUNK, D), jnp.float32),
            pltpu.VMEM((2 * S, D), jnp.float32),
            pltpu.SemaphoreType.DMA,
        ],
        compiler_params=pltpu.CompilerParams(use_tc_tiling_on_sc=False),
    )(_sc_body)
    return k(table, idx, pos2)


def kernel(sequences, table):
    idx = sequences.astype(jnp.int32).reshape(NW, NCHUNK, CHUNK)
    pos2 = _pos_encoding_2x()
    out = _embed(table, idx, pos2)
    return out.reshape(B, S, D)


# speedup vs baseline: 1.1967x; 1.1967x over previous
import functools

import jax
import jax.numpy as jnp
from jax import lax
from jax.experimental import pallas as pl
from jax.experimental.pallas import tpu as pltpu
from jax.experimental.pallas import tpu_sc as plsc

VOCAB = 1000000
D = 64
B = 1024
S = 200

NC = 2
NS = 16
NW = NC * NS
SEQ_PER_W = B // NW
ROWS_PER_W = SEQ_PER_W * S
CHUNK = 100
NCHUNK = ROWS_PER_W // CHUNK


def _pos_encoding() -> jax.Array:
    depth = D // 2
    positions = jnp.arange(S)[:, None].astype(jnp.float32)
    depths = jnp.arange(depth, dtype=jnp.float32)[None, :] / depth
    angle_rates = 1.0 / (10000.0 ** depths)
    angle_rads = positions * angle_rates
    pos = jnp.concatenate([jnp.sin(angle_rads), jnp.cos(angle_rads)], axis=-1)
    return pos.astype(jnp.float32)


def _sc_body(table_hbm, idx_hbm, pos_hbm, out_hbm,
             idx_v, pos_v, g0, g1, s0, s1,
             gsem0, gsem1, ssem0, ssem1):
    w = lax.axis_index("s") * NC + lax.axis_index("c")
    pltpu.sync_copy(idx_hbm.at[w], idx_v)
    pltpu.sync_copy(pos_hbm, pos_v)

    gbuf = (g0, g1)
    sbuf = (s0, s1)
    gsem = (gsem0, gsem1)
    ssem = (ssem0, ssem1)

    def gather(c, j):
        pltpu.async_copy(table_hbm.at[idx_v.at[c]], gbuf[j], gsem[j])

    def scatter(c, j):
        b = w * SEQ_PER_W + lax.div(c, 2)
        h = lax.rem(c, 2)
        pltpu.async_copy(sbuf[j], out_hbm.at[b, pl.ds(h * CHUNK, CHUNK)],
                         ssem[j])

    def compute(c, j):
        off = lax.rem(c, 2) * CHUNK
        gb, sb = gbuf[j], sbuf[j]

        def row_body(s2, _):
            for u in range(2):
                s = s2 * 2 + u
                p = off + s
                for d in range(D // 16):
                    sl = pl.ds(16 * d, 16)
                    sb[s, sl] = gb[s, sl] * 8.0 + pos_v[p, sl]
            return 0

        lax.fori_loop(0, CHUNK // 2, row_body, 0)

    gather(0, 0)
    gather(1, 1)

    def step(c2, _):
        for j in range(2):
            c = c2 * 2 + j
            pltpu.make_async_copy(table_hbm.at[idx_v.at[c]], gbuf[j],
                                  gsem[j]).wait()

            @pl.when(c >= 2)
            def _():
                b = w * SEQ_PER_W + lax.div(c - 2, 2)
                h = lax.rem(c - 2, 2)
                pltpu.make_async_copy(
                    sbuf[j], out_hbm.at[b, pl.ds(h * CHUNK, CHUNK)],
                    ssem[j]).wait()

            compute(c, j)

            @pl.when(c + 2 < NCHUNK)
            def _():
                gather(c + 2, j)

            scatter(c, j)
        return 0

    lax.fori_loop(0, NCHUNK // 2, step, 0)

    for j in range(2):
        c = NCHUNK - 2 + j
        b = w * SEQ_PER_W + lax.div(c, 2)
        h = lax.rem(c, 2)
        pltpu.make_async_copy(
            sbuf[j], out_hbm.at[b, pl.ds(h * CHUNK, CHUNK)], ssem[j]).wait()


@jax.jit
def _embed(table, idx, pos):
    mesh = plsc.VectorSubcoreMesh(core_axis_name="c", subcore_axis_name="s")
    k = functools.partial(
        pl.kernel,
        out_type=jax.ShapeDtypeStruct((B, S, D), jnp.float32),
        mesh=mesh,
        scratch_types=[
            pltpu.VMEM((NCHUNK, CHUNK), jnp.int32),
            pltpu.VMEM((S, D), jnp.float32),
            pltpu.VMEM((CHUNK, D), jnp.float32),
            pltpu.VMEM((CHUNK, D), jnp.float32),
            pltpu.VMEM((CHUNK, D), jnp.float32),
            pltpu.VMEM((CHUNK, D), jnp.float32),
            pltpu.SemaphoreType.DMA,
            pltpu.SemaphoreType.DMA,
            pltpu.SemaphoreType.DMA,
            pltpu.SemaphoreType.DMA,
        ],
        compiler_params=pltpu.CompilerParams(use_tc_tiling_on_sc=False),
    )(_sc_body)
    return k(table, idx, pos)


def kernel(sequences, table):
    idx = sequences.astype(jnp.int32).reshape(NW, NCHUNK, CHUNK)
    pos = _pos_encoding()
    return _embed(table, idx, pos)


# speedup vs baseline: 1.7186x; 1.4361x over previous
import functools

import jax
import jax.numpy as jnp
from jax import lax
from jax.experimental import pallas as pl
from jax.experimental.pallas import tpu as pltpu
from jax.experimental.pallas import tpu_sc as plsc

VOCAB = 1000000
D = 64
B = 1024
S = 200

NC = 2
NS = 16
NW = NC * NS
ROWS_PER_W = (B * S) // NW
CHUNK = 128
NCHUNK = ROWS_PER_W // CHUNK
NG = CHUNK // 16


def _pos_encoding() -> jax.Array:
    depth = D // 2
    positions = jnp.arange(S)[:, None].astype(jnp.float32)
    depths = jnp.arange(depth, dtype=jnp.float32)[None, :] / depth
    angle_rates = 1.0 / (10000.0 ** depths)
    angle_rads = positions * angle_rates
    pos = jnp.concatenate([jnp.sin(angle_rads), jnp.cos(angle_rads)], axis=-1)
    return pos.astype(jnp.float32)


def _sc_body(table_hbm, idx_hbm, pos_hbm, out_hbm,
             idx_v, pos_v, g0, g1, s0, s1,
             gsem0, gsem1, ssem0, ssem1):
    w = lax.axis_index("s") * NC + lax.axis_index("c")
    pltpu.sync_copy(idx_hbm.at[w], idx_v)
    pltpu.sync_copy(pos_hbm, pos_v)

    gbuf = (g0, g1)
    sbuf = (s0, s1)
    gsem = (gsem0, gsem1)
    ssem = (ssem0, ssem1)
    lane = lax.iota(jnp.int32, 16)

    def gather(c, j):
        def group(g, _):
            vec = idx_v[c, pl.ds(g * 16, 16)]
            for l in range(16):
                sel = jnp.where(lane == l, vec, 0)
                v = lax.reduce_max(sel, axes=(0,))
                t = g * 16 + l
                pltpu.async_copy(table_hbm.at[pl.ds(v, 1)],
                                 gbuf[j].at[pl.ds(t, 1)], gsem[j])
            return 0

        lax.fori_loop(0, NG, group, 0)

    def gather_wait(j):
        pltpu.make_async_copy(table_hbm.at[pl.ds(0, CHUNK)], gbuf[j],
                              gsem[j]).wait()

    def scatter(c, j):
        pltpu.async_copy(
            sbuf[j], out_hbm.at[pl.ds(w * ROWS_PER_W + c * CHUNK, CHUNK)],
            ssem[j])

    def scatter_wait(c, j):
        pltpu.make_async_copy(
            sbuf[j], out_hbm.at[pl.ds(w * ROWS_PER_W + c * CHUNK, CHUNK)],
            ssem[j]).wait()

    def compute(c, j):
        off = lax.rem(c * CHUNK, S)
        gb, sb = gbuf[j], sbuf[j]

        def row_body(s2, _):
            for u in range(2):
                s = s2 * 2 + u
                p = lax.rem(off + s, S)
                for d in range(D // 16):
                    sl = pl.ds(16 * d, 16)
                    sb[s, sl] = gb[s, sl] * 8.0 + pos_v[p, sl]
            return 0

        lax.fori_loop(0, CHUNK // 2, row_body, 0)

    gather(0, 0)
    gather(1, 1)

    def step(c2, _):
        for j in range(2):
            c = c2 * 2 + j
            gather_wait(j)

            @pl.when(c >= 2)
            def _():
                scatter_wait(c - 2, j)

            compute(c, j)

            @pl.when(c + 2 < NCHUNK)
            def _():
                gather(c + 2, j)

            scatter(c, j)
        return 0

    lax.fori_loop(0, NCHUNK // 2, step, 0)

    for j in range(2):
        scatter_wait(NCHUNK - 2 + j, j)


@jax.jit
def _embed(table, idx, pos):
    mesh = plsc.VectorSubcoreMesh(core_axis_name="c", subcore_axis_name="s")
    k = functools.partial(
        pl.kernel,
        out_type=jax.ShapeDtypeStruct((B * S, D), jnp.float32),
        mesh=mesh,
        scratch_types=[
            pltpu.VMEM((NCHUNK, CHUNK), jnp.int32),
            pltpu.VMEM((S, D), jnp.float32),
            pltpu.VMEM((CHUNK, D), jnp.float32),
            pltpu.VMEM((CHUNK, D), jnp.float32),
            pltpu.VMEM((CHUNK, D), jnp.float32),
            pltpu.VMEM((CHUNK, D), jnp.float32),
            pltpu.SemaphoreType.DMA,
            pltpu.SemaphoreType.DMA,
            pltpu.SemaphoreType.DMA,
            pltpu.SemaphoreType.DMA,
        ],
        compiler_params=pltpu.CompilerParams(needs_layout_passes=False),
    )(_sc_body)
    return k(table, idx, pos)


def kernel(sequences, table):
    idx = sequences.astype(jnp.int32).reshape(NW, NCHUNK, CHUNK)
    pos = _pos_encoding()
    out = _embed(table, idx, pos)
    return out.reshape(B, S, D)
